# Initial kernel scaffold; baseline (speedup 1.0000x reference)
import jax
import jax.numpy as jnp
from jax.experimental import pallas as pl


def kernel(user_idx, item_idx, user_table, item_table):
    raise NotImplementedError("write your pallas kernel here")


# speedup vs baseline: 1.0615x; 1.0615x over previous
import functools

import jax
import jax.numpy as jnp
from jax import lax
from jax.experimental import pallas as pl
from jax.experimental.pallas import tpu as pltpu
from jax.experimental.pallas import tpu_sc as plsc

BATCH = 16384
EMB = 128
NW = 32
CHUNK = 128
CPW = BATCH // NW // CHUNK


def _sc_dot_kernel(uidx_hbm, iidx_hbm, utab_hbm, itab_hbm, out_hbm,
                   uidx_v, iidx_v, urows_v, irows_v, outbuf_v, stage_v, sem):
    wid = lax.axis_index("s") * 2 + lax.axis_index("c")
    row0 = wid * CPW
    pltpu.sync_copy(uidx_hbm.at[pl.ds(row0, CPW)], uidx_v)
    pltpu.sync_copy(iidx_hbm.at[pl.ds(row0, CPW)], iidx_v)
    rowv = lax.iota(jnp.int32, 16)
    for c in range(CPW):
        pltpu.async_copy(utab_hbm.at[uidx_v.at[c]], urows_v, sem).wait()
        pltpu.async_copy(itab_hbm.at[iidx_v.at[c]], irows_v, sem).wait()

        def body(g, _):
            base = g * 16
            for l in range(16):
                e = base + l
                acc = urows_v[e, pl.ds(0, 16)] * irows_v[e, pl.ds(0, 16)]
                for j in range(1, EMB // 16):
                    acc = acc + urows_v[e, pl.ds(j * 16, 16)] * irows_v[e, pl.ds(j * 16, 16)]
                stage_v[l] = acc
            tot = plsc.load_gather(stage_v, [rowv, jnp.zeros((16,), jnp.int32)])
            for j in range(1, 16):
                tot = tot + plsc.load_gather(stage_v, [rowv, jnp.full((16,), j, jnp.int32)])
            outbuf_v[c, pl.ds(base, 16)] = tot
            return 0

        lax.fori_loop(0, CHUNK // 16, body, 0)
    pltpu.sync_copy(outbuf_v, out_hbm.at[pl.ds(row0, CPW)])


@jax.jit
def kernel(user_idx, item_idx, user_table, item_table):
    nrows = BATCH // CHUNK
    uidx2 = user_idx.reshape(nrows, CHUNK).astype(jnp.int32)
    iidx2 = item_idx.reshape(nrows, CHUNK).astype(jnp.int32)
    mesh = plsc.VectorSubcoreMesh(core_axis_name="c", subcore_axis_name="s")
    out = pl.kernel(
        _sc_dot_kernel,
        mesh=mesh,
        compiler_params=pltpu.CompilerParams(needs_layout_passes=False),
        out_type=jax.ShapeDtypeStruct((nrows, CHUNK), jnp.float32),
        scratch_types=[
            pltpu.VMEM((CPW, CHUNK), jnp.int32),
            pltpu.VMEM((CPW, CHUNK), jnp.int32),
            pltpu.VMEM((CHUNK, EMB), jnp.float32),
            pltpu.VMEM((CHUNK, EMB), jnp.float32),
            pltpu.VMEM((CPW, CHUNK), jnp.float32),
            pltpu.VMEM((16, 16), jnp.float32),
            pltpu.SemaphoreType.DMA,
        ],
    )(uidx2, iidx2, user_table, item_table)
    return out.reshape(BATCH)


# speedup vs baseline: 1.3223x; 1.2456x over previous
import functools

import jax
import jax.numpy as jnp
from jax import lax
from jax.experimental import pallas as pl
from jax.experimental.pallas import tpu as pltpu
from jax.experimental.pallas import tpu_sc as plsc

BATCH = 16384
EMB = 128
NW = 32
CHUNK = 128
CPW = BATCH // NW // CHUNK


def _sc_dot_kernel(uidx_hbm, iidx_hbm, utab_hbm, itab_hbm, out_hbm,
                   uidx_v, iidx_v, urows_v, irows_v, outbuf_v, stage_v,
                   sem0, sem1):
    wid = lax.axis_index("s") * 2 + lax.axis_index("c")
    row0 = wid * CPW
    pltpu.sync_copy(uidx_hbm.at[pl.ds(row0, CPW)], uidx_v)
    pltpu.sync_copy(iidx_hbm.at[pl.ds(row0, CPW)], iidx_v)
    rowv = lax.iota(jnp.int32, 16)
    sems = (sem0, sem1)

    def issue(c):
        p = c & 1
        du = pltpu.async_copy(utab_hbm.at[uidx_v.at[c]], urows_v.at[p], sems[p])
        di = pltpu.async_copy(itab_hbm.at[iidx_v.at[c]], irows_v.at[p], sems[p])
        return du, di

    pend = issue(0)
    for c in range(CPW):
        p = c & 1
        du, di = pend
        du.wait()
        di.wait()
        if c + 1 < CPW:
            pend = issue(c + 1)

        def body(g, _):
            base = g * 16
            for l in range(16):
                e = base + l
                acc = urows_v[p, e, pl.ds(0, 16)] * irows_v[p, e, pl.ds(0, 16)]
                for j in range(1, EMB // 16):
                    acc = acc + urows_v[p, e, pl.ds(j * 16, 16)] * irows_v[p, e, pl.ds(j * 16, 16)]
                stage_v[l] = acc
            tot = plsc.load_gather(stage_v, [rowv, jnp.zeros((16,), jnp.int32)])
            for j in range(1, 16):
                tot = tot + plsc.load_gather(stage_v, [rowv, jnp.full((16,), j, jnp.int32)])
            outbuf_v[c, pl.ds(base, 16)] = tot
            return 0

        lax.fori_loop(0, CHUNK // 16, body, 0)
    pltpu.sync_copy(outbuf_v, out_hbm.at[pl.ds(row0, CPW)])


@jax.jit
def kernel(user_idx, item_idx, user_table, item_table):
    nrows = BATCH // CHUNK
    uidx2 = user_idx.reshape(nrows, CHUNK).astype(jnp.int32)
    iidx2 = item_idx.reshape(nrows, CHUNK).astype(jnp.int32)
    mesh = plsc.VectorSubcoreMesh(core_axis_name="c", subcore_axis_name="s")
    out = pl.kernel(
        _sc_dot_kernel,
        mesh=mesh,
        compiler_params=pltpu.CompilerParams(needs_layout_passes=False),
        out_type=jax.ShapeDtypeStruct((nrows, CHUNK), jnp.float32),
        scratch_types=[
            pltpu.VMEM((CPW, CHUNK), jnp.int32),
            pltpu.VMEM((CPW, CHUNK), jnp.int32),
            pltpu.VMEM((2, CHUNK, EMB), jnp.float32),
            pltpu.VMEM((2, CHUNK, EMB), jnp.float32),
            pltpu.VMEM((CPW, CHUNK), jnp.float32),
            pltpu.VMEM((16, 16), jnp.float32),
            pltpu.SemaphoreType.DMA,
            pltpu.SemaphoreType.DMA,
        ],
    )(uidx2, iidx2, user_table, item_table)
    return out.reshape(BATCH)


# speedup vs baseline: 1.3348x; 1.0095x over previous
import functools

import jax
import jax.numpy as jnp
from jax import lax
from jax.experimental import pallas as pl
from jax.experimental.pallas import tpu as pltpu
from jax.experimental.pallas import tpu_sc as plsc

BATCH = 16384
EMB = 128
NW = 32
CHUNK = 128
CPW = BATCH // NW // CHUNK


def _sc_dot_kernel(uidx_hbm, iidx_hbm, utab_hbm, itab_hbm, out_hbm,
                   uidx_v, iidx_v, urows_v, irows_v, outbuf_v, stage_v,
                   sem0, sem1):
    wid = lax.axis_index("s") * 2 + lax.axis_index("c")
    row0 = wid * CPW
    pltpu.sync_copy(uidx_hbm.at[pl.ds(row0, CPW)], uidx_v)
    pltpu.sync_copy(iidx_hbm.at[pl.ds(row0, CPW)], iidx_v)
    rowv = lax.iota(jnp.int32, 16)
    sems = (sem0, sem1)

    def issue(c, p):
        pltpu.async_copy(utab_hbm.at[uidx_v.at[c]], urows_v.at[p], sems[p])
        pltpu.async_copy(itab_hbm.at[iidx_v.at[c]], irows_v.at[p], sems[p])

    issue(0, 0)

    def round_body(r, _):
        for p in (0, 1):
            c = 2 * r + p

            @pl.when(c + 1 < CPW)
            def _():
                issue(c + 1, 1 - p)

            pltpu.make_async_copy(utab_hbm.at[uidx_v.at[c]], urows_v.at[p], sems[p]).wait()
            pltpu.make_async_copy(itab_hbm.at[iidx_v.at[c]], irows_v.at[p], sems[p]).wait()

            def group_body(g, _):
                def lane_body(l, _):
                    e = g * 16 + l
                    a0 = urows_v[p, e, pl.ds(0, 16)] * irows_v[p, e, pl.ds(0, 16)]
                    a1 = urows_v[p, e, pl.ds(16, 16)] * irows_v[p, e, pl.ds(16, 16)]
                    for j in range(2, EMB // 16, 2):
                        a0 = a0 + urows_v[p, e, pl.ds(j * 16, 16)] * irows_v[p, e, pl.ds(j * 16, 16)]
                        a1 = a1 + urows_v[p, e, pl.ds(j * 16 + 16, 16)] * irows_v[p, e, pl.ds(j * 16 + 16, 16)]
                    stage_v[l] = a0 + a1
                    return 0

                lax.fori_loop(0, 16, lane_body, 0, unroll=4)
                tot = plsc.load_gather(stage_v, [rowv, jnp.zeros((16,), jnp.int32)])
                for j in range(1, 16):
                    tot = tot + plsc.load_gather(stage_v, [rowv, jnp.full((16,), j, jnp.int32)])
                outbuf_v[c, pl.ds(g * 16, 16)] = tot
                return 0

            lax.fori_loop(0, CHUNK // 16, group_body, 0)
        return 0

    lax.fori_loop(0, CPW // 2, round_body, 0)
    pltpu.sync_copy(outbuf_v, out_hbm.at[pl.ds(row0, CPW)])


@jax.jit
def kernel(user_idx, item_idx, user_table, item_table):
    nrows = BATCH // CHUNK
    uidx2 = user_idx.reshape(nrows, CHUNK).astype(jnp.int32)
    iidx2 = item_idx.reshape(nrows, CHUNK).astype(jnp.int32)
    mesh = plsc.VectorSubcoreMesh(core_axis_name="c", subcore_axis_name="s")
    out = pl.kernel(
        _sc_dot_kernel,
        mesh=mesh,
        compiler_params=pltpu.CompilerParams(needs_layout_passes=False),
        out_type=jax.ShapeDtypeStruct((nrows, CHUNK), jnp.float32),
        scratch_types=[
            pltpu.VMEM((CPW, CHUNK), jnp.int32),
            pltpu.VMEM((CPW, CHUNK), jnp.int32),
            pltpu.VMEM((2, CHUNK, EMB), jnp.float32),
            pltpu.VMEM((2, CHUNK, EMB), jnp.float32),
            pltpu.VMEM((CPW, CHUNK), jnp.float32),
            pltpu.VMEM((16, 16), jnp.float32),
            pltpu.SemaphoreType.DMA,
            pltpu.SemaphoreType.DMA,
        ],
    )(uidx2, iidx2, user_table, item_table)
    return out.reshape(BATCH)
